# baseline (speedup 1.0000x reference)
import jax
import jax.numpy as jnp
from jax import lax
from jax.experimental import pallas as pl
from jax.experimental.pallas import tpu as pltpu
from jax.experimental.pallas import tpu_sc as plsc

_D = 64
_DP = 128
_SP = 56
_NC, _NS = 2, 16
_NW = _NC * _NS
_TPB = 4
_NBUF = 2


def _emb_body(idx_hbm, table_hbm, out_hbm, idx_v, rows_v, gsem, osem):
    wid = lax.axis_index("s") * _NC + lax.axis_index("c")
    n_tok = out_hbm.shape[0] // _NW
    n_block = n_tok // _TPB
    tok_base = wid * n_tok
    pltpu.sync_copy(idx_hbm.at[pl.ds(tok_base * _SP, n_tok * _SP)], idx_v)

    def start_gathers(blk, B):
        for k in range(_TPB):
            tok = blk * _TPB + k
            pltpu.async_copy(
                table_hbm.at[idx_v.at[pl.ds(tok * _SP, 50)]],
                rows_v.at[B].at[k],
                gsem.at[B],
            )

    def wait_gathers(blk, B):
        for k in range(_TPB):
            pltpu.make_async_copy(
                table_hbm.at[idx_v.at[pl.ds((blk * _TPB + k) * _SP, 50)]],
                rows_v.at[B].at[k],
                gsem.at[B]).wait()

    def start_out(blk, B):
        for k in range(_TPB):
            pltpu.async_copy(
                rows_v.at[B].at[k],
                out_hbm.at[tok_base + blk * _TPB + k].at[pl.ds(0, 50)],
                osem.at[B])

    def wait_out(blk, B):
        for k in range(_TPB):
            pltpu.make_async_copy(
                rows_v.at[B].at[k],
                out_hbm.at[tok_base + blk * _TPB + k].at[pl.ds(0, 50)],
                osem.at[B]).wait()

    for B in range(_NBUF):
        start_gathers(B, B)

    n_group = n_block // _NBUF

    def body(g, carry):
        for B in range(_NBUF):
            blk = g * _NBUF + B
            wait_gathers(blk, B)
            start_out(blk, B)
            wait_out(blk, B)
            start_gathers(blk + _NBUF, B)
        return carry

    lax.fori_loop(0, n_group - 1, body, 0)

    for B in range(_NBUF):
        blk = (n_group - 1) * _NBUF + B
        wait_gathers(blk, B)
        start_out(blk, B)
    for B in range(_NBUF):
        blk = (n_group - 1) * _NBUF + B
        wait_out(blk, B)


def kernel(token_ids, weight):
    b, s = token_ids.shape
    idxp = jnp.pad(token_ids, ((0, 0), (0, _SP - s))).reshape(-1)
    mesh = plsc.VectorSubcoreMesh(core_axis_name="c", subcore_axis_name="s")
    n_tok = b // _NW
    out_pad = pl.kernel(
        _emb_body,
        out_type=jax.ShapeDtypeStruct((b, _SP, _DP), jnp.float32),
        mesh=mesh,
        scratch_types=[
            pltpu.VMEM((n_tok * _SP,), jnp.int32),
            pltpu.VMEM((_NBUF, _TPB, 50, _DP), jnp.float32),
            pltpu.SemaphoreType.DMA((_NBUF,)),
            pltpu.SemaphoreType.DMA((_NBUF,)),
        ],
        compiler_params=pltpu.CompilerParams(use_tc_tiling_on_sc=False),
    )(idxp, jnp.pad(weight, ((0, 0), (0, _DP - _D))))
    return out_pad[:, :s, :_D]


# speedup vs baseline: 1.0904x; 1.0904x over previous
import jax
import jax.numpy as jnp
from jax import lax
from jax.experimental import pallas as pl
from jax.experimental.pallas import tpu as pltpu
from jax.experimental.pallas import tpu_sc as plsc

_D = 64
_DP = 128
_SP = 56
_NC, _NS = 2, 16
_NW = _NC * _NS
_TPB = 4
_NBUF = 4


def _emb_body(idx_hbm, table_hbm, out_hbm, idx_v, rows_v, gsem, osem):
    wid = lax.axis_index("s") * _NC + lax.axis_index("c")
    n_tok = out_hbm.shape[0] // _NW
    n_block = n_tok // _TPB
    tok_base = wid * n_tok
    pltpu.sync_copy(idx_hbm.at[pl.ds(tok_base * _SP, n_tok * _SP)], idx_v)

    def start_gathers(blk, B):
        for k in range(_TPB):
            tok = blk * _TPB + k
            pltpu.async_copy(
                table_hbm.at[idx_v.at[pl.ds(tok * _SP, 50)]],
                rows_v.at[B].at[k],
                gsem.at[B],
            )

    def wait_gathers(blk, B):
        for k in range(_TPB):
            pltpu.make_async_copy(
                table_hbm.at[idx_v.at[pl.ds((blk * _TPB + k) * _SP, 50)]],
                rows_v.at[B].at[k],
                gsem.at[B]).wait()

    def start_out(blk, B):
        for k in range(_TPB):
            pltpu.async_copy(
                rows_v.at[B].at[k],
                out_hbm.at[tok_base + blk * _TPB + k].at[pl.ds(0, 50), pl.ds(0, _D)],
                osem.at[B])

    def wait_out(blk, B):
        for k in range(_TPB):
            pltpu.make_async_copy(
                rows_v.at[B].at[k],
                out_hbm.at[tok_base + blk * _TPB + k].at[pl.ds(0, 50), pl.ds(0, _D)],
                osem.at[B]).wait()

    for B in range(_NBUF):
        start_gathers(B, B)

    n_group = n_block // _NBUF

    def body(g, carry):
        for B in range(_NBUF):
            blk = g * _NBUF + B
            wait_gathers(blk, B)
            start_out(blk, B)
            wait_out(blk, B)
            start_gathers(blk + _NBUF, B)
        return carry

    lax.fori_loop(0, n_group - 1, body, 0)

    for B in range(_NBUF):
        blk = (n_group - 1) * _NBUF + B
        wait_gathers(blk, B)
        start_out(blk, B)
    for B in range(_NBUF):
        blk = (n_group - 1) * _NBUF + B
        wait_out(blk, B)


def kernel(token_ids, weight):
    b, s = token_ids.shape
    idxp = jnp.pad(token_ids, ((0, 0), (0, _SP - s))).reshape(-1)
    mesh = plsc.VectorSubcoreMesh(core_axis_name="c", subcore_axis_name="s")
    n_tok = b // _NW
    out_pad = pl.kernel(
        _emb_body,
        out_type=jax.ShapeDtypeStruct((b, _SP, _DP), jnp.float32),
        mesh=mesh,
        scratch_types=[
            pltpu.VMEM((n_tok * _SP,), jnp.int32),
            pltpu.VMEM((_NBUF, _TPB, 50, _D), jnp.float32),
            pltpu.SemaphoreType.DMA((_NBUF,)),
            pltpu.SemaphoreType.DMA((_NBUF,)),
        ],
        compiler_params=pltpu.CompilerParams(use_tc_tiling_on_sc=False),
    )(idxp, weight)
    return out_pad[:, :s, :_D]


# speedup vs baseline: 1.0966x; 1.0057x over previous
import jax
import jax.numpy as jnp
from jax import lax
from jax.experimental import pallas as pl
from jax.experimental.pallas import tpu as pltpu
from jax.experimental.pallas import tpu_sc as plsc

_D = 64
_DP = 128
_SP = 56
_NC, _NS = 2, 16
_NW = _NC * _NS
_TPB = 8
_NBUF = 2


def _emb_body(idx_hbm, table_hbm, out_hbm, idx_v, rows_v, gsem, osem):
    wid = lax.axis_index("s") * _NC + lax.axis_index("c")
    n_tok = out_hbm.shape[0] // _NW
    n_block = n_tok // _TPB
    tok_base = wid * n_tok
    pltpu.sync_copy(idx_hbm.at[pl.ds(tok_base * _SP, n_tok * _SP)], idx_v)

    def start_gathers(blk, B):
        for k in range(_TPB):
            tok = blk * _TPB + k
            pltpu.async_copy(
                table_hbm.at[idx_v.at[pl.ds(tok * _SP, 50)]],
                rows_v.at[B].at[k],
                gsem.at[B],
            )

    def wait_gathers(blk, B):
        for k in range(_TPB):
            pltpu.make_async_copy(
                table_hbm.at[idx_v.at[pl.ds((blk * _TPB + k) * _SP, 50)]],
                rows_v.at[B].at[k],
                gsem.at[B]).wait()

    def start_out(blk, B):
        for k in range(_TPB):
            pltpu.async_copy(
                rows_v.at[B].at[k],
                out_hbm.at[tok_base + blk * _TPB + k].at[pl.ds(0, 50), pl.ds(0, _D)],
                osem.at[B])

    def wait_out(blk, B):
        for k in range(_TPB):
            pltpu.make_async_copy(
                rows_v.at[B].at[k],
                out_hbm.at[tok_base + blk * _TPB + k].at[pl.ds(0, 50), pl.ds(0, _D)],
                osem.at[B]).wait()

    for B in range(_NBUF):
        start_gathers(B, B)

    n_group = n_block // _NBUF

    def body(g, carry):
        for B in range(_NBUF):
            blk = g * _NBUF + B
            wait_gathers(blk, B)
            start_out(blk, B)
            wait_out(blk, B)
            start_gathers(blk + _NBUF, B)
        return carry

    lax.fori_loop(0, n_group - 1, body, 0)

    for B in range(_NBUF):
        blk = (n_group - 1) * _NBUF + B
        wait_gathers(blk, B)
        start_out(blk, B)
    for B in range(_NBUF):
        blk = (n_group - 1) * _NBUF + B
        wait_out(blk, B)


def kernel(token_ids, weight):
    b, s = token_ids.shape
    idxp = jnp.pad(token_ids, ((0, 0), (0, _SP - s))).reshape(-1)
    mesh = plsc.VectorSubcoreMesh(core_axis_name="c", subcore_axis_name="s")
    n_tok = b // _NW
    out_pad = pl.kernel(
        _emb_body,
        out_type=jax.ShapeDtypeStruct((b, _SP, _DP), jnp.float32),
        mesh=mesh,
        scratch_types=[
            pltpu.VMEM((n_tok * _SP,), jnp.int32),
            pltpu.VMEM((_NBUF, _TPB, 50, _D), jnp.float32),
            pltpu.SemaphoreType.DMA((_NBUF,)),
            pltpu.SemaphoreType.DMA((_NBUF,)),
        ],
        compiler_params=pltpu.CompilerParams(use_tc_tiling_on_sc=False),
    )(idxp, weight)
    return out_pad[:, :s, :_D]
